# baseline (speedup 1.0000x reference)
import functools

import jax
import jax.numpy as jnp
from jax import lax
from jax.experimental import pallas as pl
from jax.experimental.pallas import tpu as pltpu
from jax.experimental.pallas import tpu_sc as plsc

_NW = 32
_C = 256
_D_PAD = 128


def _tc_format(table_t, v, d):
    blk = 512
    grid = (v + blk - 1) // blk

    def body(t_ref, o_ref):
        x = t_ref[...]
        xt = x.T
        o_ref[...] = jnp.concatenate([xt, xt], axis=1)

    return pl.pallas_call(
        body,
        grid=(grid,),
        in_specs=[pl.BlockSpec((d, blk), lambda i: (0, i))],
        out_specs=pl.BlockSpec((blk, _D_PAD), lambda i: (i, 0)),
        out_shape=jax.ShapeDtypeStruct((v, _D_PAD), table_t.dtype),
    )(table_t)


def _sc_gather(table_pad, idx, n_idx, d):
    per_w = n_idx // _NW
    steps = per_w // _C
    mesh = plsc.VectorSubcoreMesh(core_axis_name="c", subcore_axis_name="s")

    @functools.partial(
        pl.kernel,
        out_type=jax.ShapeDtypeStruct((n_idx, _D_PAD), table_pad.dtype),
        mesh=mesh,
        compiler_params=pltpu.CompilerParams(use_tc_tiling_on_sc=True),
        scratch_types=[
            pltpu.VMEM((per_w,), jnp.int32),
            pltpu.VMEM((_C, _D_PAD), jnp.float32),
            pltpu.VMEM((_C, _D_PAD), jnp.float32),
            pltpu.SemaphoreType.DMA,
            pltpu.SemaphoreType.DMA,
            pltpu.SemaphoreType.DMA,
            pltpu.SemaphoreType.DMA,
        ],
    )
    def k(table_hbm, idx_hbm, out_hbm, idx_v, r0, r1, g0, g1, w0, w1):
        wid = lax.axis_index("s") * 2 + lax.axis_index("c")
        base = wid * per_w
        pltpu.sync_copy(idx_hbm.at[pl.ds(base, per_w)], idx_v)
        rows = (r0, r1)
        gsem = (g0, g1)
        wsem = (w0, w1)

        def fire_gather(j, b):
            pltpu.async_copy(
                table_hbm.at[idx_v.at[pl.ds(j * _C, _C)]], rows[b], gsem[b]
            )

        def wait_gather(b):
            pltpu.make_async_copy(
                table_hbm.at[idx_v.at[pl.ds(0, _C)]], rows[b], gsem[b]
            ).wait()

        def fire_write(j, b):
            pltpu.async_copy(
                rows[b],
                out_hbm.at[pl.ds(base + j * _C, _C)],
                wsem[b],
            )

        def wait_write(b):
            pltpu.make_async_copy(
                rows[b],
                out_hbm.at[pl.ds(base, _C)],
                wsem[b],
            ).wait()

        fire_gather(0, 0)

        @pl.loop(0, steps, step=2)
        def _(g):
            wait_gather(0)

            @pl.when(g > 0)
            def _():
                wait_write(1)

            fire_gather(g + 1, 1)
            fire_write(g, 0)

            wait_gather(1)
            wait_write(0)

            @pl.when(g + 2 < steps)
            def _():
                fire_gather(g + 2, 0)

            fire_write(g + 1, 1)

        wait_write(1)

    return k(table_pad, idx)


def kernel(x, table):
    b, s = x.shape
    v, d = table.shape
    n = b * s
    table_pad = _tc_format(table.T, v, d)
    idx = x.reshape(n).astype(jnp.int32)
    out = _sc_gather(table_pad, idx, n, d)
    return out[:, :d].reshape(b, s, d)


# speedup vs baseline: 2.1551x; 2.1551x over previous
import functools

import jax
import jax.numpy as jnp
from jax import lax
from jax.experimental import pallas as pl
from jax.experimental.pallas import tpu as pltpu
from jax.experimental.pallas import tpu_sc as plsc

_NW = 32
_C = 256
_D_PAD = 128


def _tc_format(table_t, v, d):
    blk = 8192
    grid = (v + blk - 1) // blk

    def body(t_ref, o_ref):
        x = t_ref[...]
        xt = x.T
        o_ref[...] = jnp.concatenate([xt, xt], axis=1)

    return pl.pallas_call(
        body,
        grid=(grid,),
        in_specs=[pl.BlockSpec((d, blk), lambda i: (0, i))],
        out_specs=pl.BlockSpec((blk, _D_PAD), lambda i: (i, 0)),
        out_shape=jax.ShapeDtypeStruct((v, _D_PAD), table_t.dtype),
    )(table_t)


def _sc_gather(table_pad, idx, n_idx, d):
    per_w = n_idx // _NW
    steps = per_w // _C
    mesh = plsc.VectorSubcoreMesh(core_axis_name="c", subcore_axis_name="s")

    @functools.partial(
        pl.kernel,
        out_type=jax.ShapeDtypeStruct((n_idx, _D_PAD), table_pad.dtype),
        mesh=mesh,
        compiler_params=pltpu.CompilerParams(use_tc_tiling_on_sc=True),
        scratch_types=[
            pltpu.VMEM((per_w,), jnp.int32),
            pltpu.VMEM((_C, _D_PAD), jnp.float32),
            pltpu.VMEM((_C, _D_PAD), jnp.float32),
            pltpu.SemaphoreType.DMA,
            pltpu.SemaphoreType.DMA,
            pltpu.SemaphoreType.DMA,
            pltpu.SemaphoreType.DMA,
        ],
    )
    def k(table_hbm, idx_hbm, out_hbm, idx_v, r0, r1, g0, g1, w0, w1):
        wid = lax.axis_index("s") * 2 + lax.axis_index("c")
        base = wid * per_w
        pltpu.sync_copy(idx_hbm.at[pl.ds(base, per_w)], idx_v)
        rows = (r0, r1)
        gsem = (g0, g1)
        wsem = (w0, w1)

        def fire_gather(j, b):
            pltpu.async_copy(
                table_hbm.at[idx_v.at[pl.ds(j * _C, _C)]], rows[b], gsem[b]
            )

        def wait_gather(b):
            pltpu.make_async_copy(
                table_hbm.at[idx_v.at[pl.ds(0, _C)]], rows[b], gsem[b]
            ).wait()

        def fire_write(j, b):
            pltpu.async_copy(
                rows[b],
                out_hbm.at[pl.ds(base + j * _C, _C)],
                wsem[b],
            )

        def wait_write(b):
            pltpu.make_async_copy(
                rows[b],
                out_hbm.at[pl.ds(base, _C)],
                wsem[b],
            ).wait()

        fire_gather(0, 0)

        @pl.loop(0, steps, step=2)
        def _(g):
            wait_gather(0)

            @pl.when(g > 0)
            def _():
                wait_write(1)

            fire_gather(g + 1, 1)
            fire_write(g, 0)

            wait_gather(1)
            wait_write(0)

            @pl.when(g + 2 < steps)
            def _():
                fire_gather(g + 2, 0)

            fire_write(g + 1, 1)

        wait_write(1)

    return k(table_pad, idx)


def kernel(x, table):
    b, s = x.shape
    v, d = table.shape
    n = b * s
    table_pad = _tc_format(table.T, v, d)
    idx = x.reshape(n).astype(jnp.int32)
    out = _sc_gather(table_pad, idx, n, d)
    return out[:, :d].reshape(b, s, d)


# speedup vs baseline: 2.3109x; 1.0723x over previous
import functools

import jax
import jax.numpy as jnp
from jax import lax
from jax.experimental import pallas as pl
from jax.experimental.pallas import tpu as pltpu
from jax.experimental.pallas import tpu_sc as plsc

_NW = 32
_C = 256
_D_PAD = 128


def _tc_format(table_t, v, d):
    blk = 8192
    grid = (v + blk - 1) // blk

    def body(t_ref, o_ref):
        o_ref[:, :d] = t_ref[...].T

    return pl.pallas_call(
        body,
        grid=(grid,),
        in_specs=[pl.BlockSpec((d, blk), lambda i: (0, i))],
        out_specs=pl.BlockSpec((blk, _D_PAD), lambda i: (i, 0)),
        out_shape=jax.ShapeDtypeStruct((v, _D_PAD), table_t.dtype),
    )(table_t)


def _sc_gather(table_pad, idx, n_idx, d):
    per_w = n_idx // _NW
    steps = per_w // _C
    mesh = plsc.VectorSubcoreMesh(core_axis_name="c", subcore_axis_name="s")

    @functools.partial(
        pl.kernel,
        out_type=jax.ShapeDtypeStruct((n_idx, _D_PAD), table_pad.dtype),
        mesh=mesh,
        compiler_params=pltpu.CompilerParams(use_tc_tiling_on_sc=True),
        scratch_types=[
            pltpu.VMEM((per_w,), jnp.int32),
            pltpu.VMEM((_C, _D_PAD), jnp.float32),
            pltpu.VMEM((_C, _D_PAD), jnp.float32),
            pltpu.SemaphoreType.DMA,
            pltpu.SemaphoreType.DMA,
            pltpu.SemaphoreType.DMA,
            pltpu.SemaphoreType.DMA,
        ],
    )
    def k(table_hbm, idx_hbm, out_hbm, idx_v, r0, r1, g0, g1, w0, w1):
        wid = lax.axis_index("s") * 2 + lax.axis_index("c")
        base = wid * per_w
        pltpu.sync_copy(idx_hbm.at[pl.ds(base, per_w)], idx_v)
        rows = (r0, r1)
        gsem = (g0, g1)
        wsem = (w0, w1)

        def fire_gather(j, b):
            pltpu.async_copy(
                table_hbm.at[idx_v.at[pl.ds(j * _C, _C)]], rows[b], gsem[b]
            )

        def wait_gather(b):
            pltpu.make_async_copy(
                table_hbm.at[idx_v.at[pl.ds(0, _C)]], rows[b], gsem[b]
            ).wait()

        def fire_write(j, b):
            pltpu.async_copy(
                rows[b],
                out_hbm.at[pl.ds(base + j * _C, _C)],
                wsem[b],
            )

        def wait_write(b):
            pltpu.make_async_copy(
                rows[b],
                out_hbm.at[pl.ds(base, _C)],
                wsem[b],
            ).wait()

        fire_gather(0, 0)

        @pl.loop(0, steps, step=2)
        def _(g):
            wait_gather(0)

            @pl.when(g > 0)
            def _():
                wait_write(1)

            fire_gather(g + 1, 1)
            fire_write(g, 0)

            wait_gather(1)
            wait_write(0)

            @pl.when(g + 2 < steps)
            def _():
                fire_gather(g + 2, 0)

            fire_write(g + 1, 1)

        wait_write(1)

    return k(table_pad, idx)


def kernel(x, table):
    b, s = x.shape
    v, d = table.shape
    n = b * s
    table_pad = _tc_format(table.T, v, d)
    idx = x.reshape(n).astype(jnp.int32)
    out = _sc_gather(table_pad, idx, n, d)
    return out[:, :d].reshape(b, s, d)
